# baseline (speedup 1.0000x reference)
import jax
import jax.numpy as jnp
from jax import lax
from jax.experimental import pallas as pl
from jax.experimental.pallas import tpu as pltpu

_BN_EPS = 1e-5
_C = 3

_PAIRS = ((0, 0), (0, 1), (0, 2), (1, 1), (1, 2), (2, 2))
_NSTAT = _C + len(_PAIRS)
_LANE = 128
_SUB = 8
_NPARAM = 2 * _C + 3
_TARGET_BLOCK_BYTES = 6 * 1024 * 1024


def _round_up(v, m):
    return -(-v // m) * m


def _part_sum(a):
    lane = a.shape[-1]
    rows = a.size // lane
    return a.reshape(rows // _SUB, _SUB, lane).sum(axis=0)


def _plan_tiles(rows, n):
    per_sample = _C * rows * _LANE * 4
    if per_sample <= _TARGET_BLOCK_BYTES:
        s = rows
        nb = 1
        want = max(1, _TARGET_BLOCK_BYTES // per_sample)
        for d in range(1, n + 1):
            if n % d == 0 and d <= want:
                nb = d
    else:
        nb = 1
        s = _SUB
        cap = _TARGET_BLOCK_BYTES // (_C * _LANE * 4)
        for cand in range(_SUB, rows + 1, _SUB):
            if rows % cand == 0 and cand <= cap:
                s = cand
    return nb, s


def _fused_forward(x_nchw, w1, b1, gamma, beta, w2, b2):
    del b1
    N, c_in, H, W = x_nchw.shape
    assert c_in == _C
    HW = H * W
    M = N * HW
    inv_m = 1.0 / float(M)

    HWp = _round_up(HW, _LANE * _SUB)
    ROWS = HWp // _LANE

    x3 = x_nchw.reshape(N, _C, HW).astype(jnp.float32)
    if HWp != HW:
        x3 = jnp.pad(x3, ((0, 0), (0, 0), (0, HWp - HW)))
    x4 = x3.reshape(N, _C, ROWS, _LANE)

    Nb, S = _plan_tiles(ROWS, N)
    tn = N // Nb
    tr = ROWS // S
    T = tn * tr
    resident = N * _C * ROWS * _LANE * 4 <= 40 * 1024 * 1024

    w1f = w1.astype(jnp.float32)
    w2f = w2.astype(jnp.float32)
    params = jnp.concatenate(
        [w1f, w2f,
         gamma.astype(jnp.float32)[:, None],
         beta.astype(jnp.float32)[:, None],
         b2.astype(jnp.float32)[:, None]], axis=1)

    def body(p_ref, x_ref, o_ref, acc_ref, wb_ref, xbuf_ref):
        ph = pl.program_id(0)
        n = pl.program_id(1)
        r = pl.program_id(2)
        t = n * tr + r

        @pl.when(jnp.logical_and(ph == 0, t == 0))
        def _init():
            acc_ref[...] = jnp.zeros_like(acc_ref)

        @pl.when(ph == 0)
        def _stats():
            xv = x_ref[...]
            if resident:
                xbuf_ref[pl.ds(n * Nb, Nb), :, pl.ds(r * S, S), :] = xv
            xs = [xv[:, c:c + 1, :, :] for c in range(_C)]
            parts = [_part_sum(xs[c]) for c in range(_C)]
            parts += [_part_sum(xs[i] * xs[j]) for (i, j) in _PAIRS]
            acc_ref[...] += jnp.stack(parts, axis=0)

        @pl.when(jnp.logical_and(ph == 0, t == T - 1))
        def _fold():
            tot = [jnp.sum(acc_ref[k]) for k in range(_NSTAT)]
            mean = [tot[c] * inv_m for c in range(_C)]
            exx = {}
            for k, (i, j) in enumerate(_PAIRS):
                exx[(i, j)] = tot[_C + k] * inv_m
                exx[(j, i)] = exx[(i, j)]
            cov = [[exx[(i, j)] - mean[i] * mean[j] for j in range(_C)]
                   for i in range(_C)]
            w1s = [[p_ref[i, j] for j in range(_C)] for i in range(_C)]
            w2s = [[p_ref[i, _C + j] for j in range(_C)] for i in range(_C)]
            g = []
            for c in range(_C):
                vh = sum(w1s[c][i] * cov[i][j] * w1s[c][j]
                         for i in range(_C) for j in range(_C))
                vh = jnp.maximum(vh, 0.0) + _BN_EPS
                rs = lax.rsqrt(jnp.full((1, _LANE), vh, jnp.float32))[0, 0]
                g.append(p_ref[c, 2 * _C] * rs)
            for c in range(_C):
                for j in range(_C):
                    wb_ref[c, j] = sum(w2s[c][k] * g[k] * w1s[k][j]
                                       for k in range(_C))
                mh = [sum(w1s[k][i] * mean[i] for i in range(_C))
                      for k in range(_C)]
                wb_ref[c, _C] = p_ref[c, 2 * _C + 2] + sum(
                    w2s[c][k] * (p_ref[k, 2 * _C + 1] - g[k] * mh[k])
                    for k in range(_C))

        if resident:
            xv = jnp.where(
                ph == 0, x_ref[...],
                xbuf_ref[pl.ds(n * Nb, Nb), :, pl.ds(r * S, S), :])
        else:
            xv = x_ref[...]
        xs = [xv[:, c:c + 1, :, :] for c in range(_C)]
        for c in range(_C):
            o_ref[:, c:c + 1, :, :] = (
                wb_ref[c, 0] * xs[0] + wb_ref[c, 1] * xs[1]
                + wb_ref[c, 2] * xs[2] + wb_ref[c, _C])

    if resident:
        x_spec = pl.BlockSpec(
            (Nb, _C, S, _LANE),
            lambda p, n, r: (jnp.where(p == 0, n, tn - 1), 0,
                             jnp.where(p == 0, r, tr - 1), 0))
    else:
        x_spec = pl.BlockSpec((Nb, _C, S, _LANE),
                              lambda p, n, r: (n, 0, r, 0))
    o_spec = pl.BlockSpec(
        (Nb, _C, S, _LANE),
        lambda p, n, r: (jnp.where(p == 0, 0, n), 0,
                         jnp.where(p == 0, 0, r), 0))
    p_spec = pl.BlockSpec((_C, _NPARAM), lambda p, n, r: (0, 0),
                          memory_space=pltpu.MemorySpace.SMEM)

    out4 = pl.pallas_call(
        body,
        out_shape=jax.ShapeDtypeStruct((N, _C, ROWS, _LANE), jnp.float32),
        grid=(2, tn, tr),
        in_specs=[p_spec, x_spec],
        out_specs=o_spec,
        scratch_shapes=[pltpu.VMEM((_NSTAT, _SUB, _LANE), jnp.float32),
                        pltpu.SMEM((_C, _C + 1), jnp.float32),
                        pltpu.VMEM((N, _C, ROWS, _LANE) if resident
                                   else (1, 1, _SUB, _LANE), jnp.float32)],
        compiler_params=pltpu.CompilerParams(
            dimension_semantics=("arbitrary", "arbitrary", "arbitrary"),
            vmem_limit_bytes=64 * 1024 * 1024),
        cost_estimate=pl.CostEstimate(
            flops=33 * M, transcendentals=0, bytes_accessed=12 * _C * M),
    )(params, x4)

    out3 = out4.reshape(N, _C, HWp)
    if HWp != HW:
        out3 = out3[:, :, :HW]
    return out3.reshape(N, _C, H, W)


def kernel(x_nchw, w1, b1, gamma, beta, w2, b2):
    return _fused_forward(x_nchw, w1, b1, gamma, beta, w2, b2)


# speedup vs baseline: 1.0329x; 1.0329x over previous
import jax
import jax.numpy as jnp
from jax import lax
from jax.experimental import pallas as pl
from jax.experimental.pallas import tpu as pltpu

_BN_EPS = 1e-5
_C = 3

_PAIRS = ((0, 0), (0, 1), (0, 2), (1, 1), (1, 2), (2, 2))
_NSTAT = _C + len(_PAIRS)
_LANE = 128
_SUB = 8
_NPARAM = 2 * _C + 3
_TARGET_BLOCK_BYTES = 6 * 1024 * 1024
_RESIDENT_LIMIT = 38 * 1024 * 1024


def _round_up(v, m):
    return -(-v // m) * m


def _part_sum(a):
    lane = a.shape[-1]
    rows = a.size // lane
    return a.reshape(rows // _SUB, _SUB, lane).sum(axis=0)


def _plan_tiles(rows, n):
    per_sample = _C * rows * _LANE * 4
    if per_sample <= _TARGET_BLOCK_BYTES:
        s = rows
        nb = 1
        want = max(1, _TARGET_BLOCK_BYTES // per_sample)
        for d in range(1, n + 1):
            if n % d == 0 and d <= want:
                nb = d
    else:
        nb = 1
        s = _SUB
        cap = _TARGET_BLOCK_BYTES // (_C * _LANE * 4)
        for cand in range(_SUB, rows + 1, _SUB):
            if rows % cand == 0 and cand <= cap:
                s = cand
    return nb, s


def _tile_stats(xv):
    xs = [xv[:, c:c + 1, :, :] for c in range(_C)]
    parts = [_part_sum(xs[c]) for c in range(_C)]
    parts += [_part_sum(xs[i] * xs[j]) for (i, j) in _PAIRS]
    return parts


def _fold_affine(tot, p_ref, inv_m):
    mean = [tot[c] * inv_m for c in range(_C)]
    exx = {}
    for k, (i, j) in enumerate(_PAIRS):
        exx[(i, j)] = tot[_C + k] * inv_m
        exx[(j, i)] = exx[(i, j)]
    cov = [[exx[(i, j)] - mean[i] * mean[j] for j in range(_C)]
           for i in range(_C)]
    w1s = [[p_ref[i, j] for j in range(_C)] for i in range(_C)]
    w2s = [[p_ref[i, _C + j] for j in range(_C)] for i in range(_C)]
    g = []
    for c in range(_C):
        vh = sum(w1s[c][i] * cov[i][j] * w1s[c][j]
                 for i in range(_C) for j in range(_C))
        vh = jnp.maximum(vh, 0.0) + _BN_EPS
        rs = lax.rsqrt(jnp.full((1, _LANE), vh, jnp.float32))[0, 0]
        g.append(p_ref[c, 2 * _C] * rs)
    mh = [sum(w1s[k][i] * mean[i] for i in range(_C)) for k in range(_C)]
    w_eff = [[sum(w2s[c][k] * g[k] * w1s[k][j] for k in range(_C))
              for j in range(_C)] for c in range(_C)]
    b_eff = [p_ref[c, 2 * _C + 2]
             + sum(w2s[c][k] * (p_ref[k, 2 * _C + 1] - g[k] * mh[k])
                   for k in range(_C))
             for c in range(_C)]
    return w_eff, b_eff


def _apply_affine(xv, w_eff, b_eff):
    xs = [xv[:, c:c + 1, :, :] for c in range(_C)]
    return jnp.concatenate(
        [w_eff[c][0] * xs[0] + w_eff[c][1] * xs[1]
         + w_eff[c][2] * xs[2] + b_eff[c] for c in range(_C)], axis=1)


def _forward_resident(x4, params, inv_m, Nb, S, tn, tr):
    N, _, ROWS, _ = x4.shape
    T = tn * tr
    tiles = [divmod(t, tr) for t in range(T)]

    def x_slice(ref, n, r):
        return ref.at[pl.ds(n * Nb, Nb), slice(None), pl.ds(r * S, S),
                      slice(None)]

    def body(p_ref, x_hbm, o_hbm, xbuf, obuf, in_sems, out_sems):
        for t, (n, r) in enumerate(tiles):
            pltpu.make_async_copy(
                x_slice(x_hbm, n, r), x_slice(xbuf, n, r),
                in_sems.at[t]).start()

        tot9 = None
        for t, (n, r) in enumerate(tiles):
            pltpu.make_async_copy(
                x_slice(x_hbm, n, r), x_slice(xbuf, n, r),
                in_sems.at[t]).wait()
            parts = _tile_stats(
                xbuf[pl.ds(n * Nb, Nb), :, pl.ds(r * S, S), :])
            tot9 = parts if tot9 is None else [
                a + b for a, b in zip(tot9, parts)]

        tot = [jnp.sum(v) for v in tot9]
        w_eff, b_eff = _fold_affine(tot, p_ref, inv_m)

        def out_copy(slot, t):
            n, r = tiles[t]
            return pltpu.make_async_copy(
                obuf.at[slot], x_slice(o_hbm, n, r), out_sems.at[slot])

        for t, (n, r) in enumerate(tiles):
            slot = t % 2
            if t >= 2:
                out_copy(slot, t - 2).wait()
            obuf[slot] = _apply_affine(
                xbuf[pl.ds(n * Nb, Nb), :, pl.ds(r * S, S), :],
                w_eff, b_eff)
            out_copy(slot, t).start()
        for t in range(max(0, T - 2), T):
            out_copy(t % 2, t).wait()

    return pl.pallas_call(
        body,
        out_shape=jax.ShapeDtypeStruct((N, _C, ROWS, _LANE), jnp.float32),
        in_specs=[pl.BlockSpec(memory_space=pltpu.MemorySpace.SMEM),
                  pl.BlockSpec(memory_space=pl.MemorySpace.ANY)],
        out_specs=pl.BlockSpec(memory_space=pl.MemorySpace.ANY),
        scratch_shapes=[
            pltpu.VMEM((N, _C, ROWS, _LANE), jnp.float32),
            pltpu.VMEM((2, Nb, _C, S, _LANE), jnp.float32),
            pltpu.SemaphoreType.DMA((T,)),
            pltpu.SemaphoreType.DMA((2,)),
        ],
        compiler_params=pltpu.CompilerParams(
            vmem_limit_bytes=64 * 1024 * 1024),
    )(params, x4)


def _forward_streaming(x4, params, inv_m, Nb, S, tn, tr):
    N, _, ROWS, _ = x4.shape
    T = tn * tr

    def body(p_ref, x_ref, o_ref, acc_ref, wb_ref):
        ph = pl.program_id(0)
        t = pl.program_id(1) * tr + pl.program_id(2)

        @pl.when(jnp.logical_and(ph == 0, t == 0))
        def _init():
            acc_ref[...] = jnp.zeros_like(acc_ref)

        @pl.when(ph == 0)
        def _stats():
            acc_ref[...] += jnp.stack(_tile_stats(x_ref[...]), axis=0)

        @pl.when(jnp.logical_and(ph == 0, t == T - 1))
        def _fold():
            tot = [jnp.sum(acc_ref[k]) for k in range(_NSTAT)]
            w_eff, b_eff = _fold_affine(tot, p_ref, inv_m)
            for c in range(_C):
                for j in range(_C):
                    wb_ref[c, j] = w_eff[c][j]
                wb_ref[c, _C] = b_eff[c]

        @pl.when(ph == 1)
        def _apply():
            w_eff = [[wb_ref[c, j] for j in range(_C)] for c in range(_C)]
            b_eff = [wb_ref[c, _C] for c in range(_C)]
            o_ref[...] = _apply_affine(x_ref[...], w_eff, b_eff)

    x_spec = pl.BlockSpec((Nb, _C, S, _LANE), lambda p, n, r: (n, 0, r, 0))
    o_spec = pl.BlockSpec(
        (Nb, _C, S, _LANE),
        lambda p, n, r: (jnp.where(p == 0, 0, n), 0,
                         jnp.where(p == 0, 0, r), 0))
    p_spec = pl.BlockSpec((_C, _NPARAM), lambda p, n, r: (0, 0),
                          memory_space=pltpu.MemorySpace.SMEM)

    return pl.pallas_call(
        body,
        out_shape=jax.ShapeDtypeStruct((N, _C, ROWS, _LANE), jnp.float32),
        grid=(2, tn, tr),
        in_specs=[p_spec, x_spec],
        out_specs=o_spec,
        scratch_shapes=[pltpu.VMEM((_NSTAT, _SUB, _LANE), jnp.float32),
                        pltpu.SMEM((_C, _C + 1), jnp.float32)],
        compiler_params=pltpu.CompilerParams(
            dimension_semantics=("arbitrary", "arbitrary", "arbitrary"),
            vmem_limit_bytes=64 * 1024 * 1024),
    )(params, x4)


def _fused_forward(x_nchw, w1, b1, gamma, beta, w2, b2):
    del b1
    N, c_in, H, W = x_nchw.shape
    assert c_in == _C
    HW = H * W
    M = N * HW
    inv_m = 1.0 / float(M)

    HWp = _round_up(HW, _LANE * _SUB)
    ROWS = HWp // _LANE

    x3 = x_nchw.reshape(N, _C, HW).astype(jnp.float32)
    if HWp != HW:
        x3 = jnp.pad(x3, ((0, 0), (0, 0), (0, HWp - HW)))
    x4 = x3.reshape(N, _C, ROWS, _LANE)

    Nb, S = _plan_tiles(ROWS, N)
    tn = N // Nb
    tr = ROWS // S

    params = jnp.concatenate(
        [w1.astype(jnp.float32), w2.astype(jnp.float32),
         gamma.astype(jnp.float32)[:, None],
         beta.astype(jnp.float32)[:, None],
         b2.astype(jnp.float32)[:, None]], axis=1)

    if N * _C * ROWS * _LANE * 4 <= _RESIDENT_LIMIT:
        out4 = _forward_resident(x4, params, inv_m, Nb, S, tn, tr)
    else:
        out4 = _forward_streaming(x4, params, inv_m, Nb, S, tn, tr)

    out3 = out4.reshape(N, _C, HWp)
    if HWp != HW:
        out3 = out3[:, :, :HW]
    return out3.reshape(N, _C, H, W)


def kernel(x_nchw, w1, b1, gamma, beta, w2, b2):
    return _fused_forward(x_nchw, w1, b1, gamma, beta, w2, b2)
